# Initial kernel scaffold; baseline (speedup 1.0000x reference)
import jax
import jax.numpy as jnp
from jax.experimental import pallas as pl


def kernel(h_e, x_orig, Wq, bq, Wk, bk, Wv, bv, W1, b1, W2, b2):
    raise NotImplementedError("write your pallas kernel here")


# speedup vs baseline: 7.1623x; 7.1623x over previous
import functools
import math

import jax
import jax.numpy as jnp
from jax import lax
from jax.experimental import pallas as pl
from jax.experimental.pallas import tpu as pltpu
import jax.experimental.pallas.tpu_sc as plsc

_TAU = 4
_K = 8
_ROWS = 256


def _qkv_body(h_last_ref, hist_ref, wq_ref, bq_ref, wk_ref, bk_ref,
              wv_ref, bv_ref, q_ref, kv_ref):
    h_last = h_last_ref[0]
    hist = hist_ref[0]
    q_ref[0] = jnp.dot(h_last, wq_ref[...],
                       preferred_element_type=jnp.float32) + bq_ref[...]
    kfeat = jnp.dot(hist, wk_ref[...],
                    preferred_element_type=jnp.float32) + bk_ref[...]
    vfeat = jnp.dot(hist, wv_ref[...],
                    preferred_element_type=jnp.float32) + bv_ref[...]
    kv_ref[0] = jnp.concatenate([kfeat, vfeat], axis=-1)


def _topk_body(windc_ref, windr_ref, fidx_ref, *, n, tau, k):
    b = pl.program_id(0)
    wcx = windc_ref[0, :, 0:1]
    wcy = windc_ref[0, :, 1:2]
    wrx = windr_ref[0, 0:1, :]
    wry = windr_ref[0, 1:2, :]
    dx = wcx - wrx
    dy = wcy - wry
    d2 = dx * dx + dy * dy
    rows = d2.shape[0]
    iota = lax.broadcasted_iota(jnp.int32, (rows, n), 1)
    cols = []
    for _ in range(k):
        m = jnp.min(d2, axis=1, keepdims=True)
        eq = d2 == m
        j = jnp.min(jnp.where(eq, iota, n), axis=1, keepdims=True)
        hit = iota == j
        d2 = jnp.where(hit, jnp.inf, d2)
        base = j + b * (tau * n)
        for t in range(tau):
            cols.append(base + t * n)
    fidx_ref[0] = jnp.concatenate(cols, axis=1)


def _sc_gather_body(kv_hbm, idx_hbm, out_hbm, idx_v, rows_v, sem, *,
                    rows_per_worker, chunk, num_cores):
    wid = lax.axis_index("s") * num_cores + lax.axis_index("c")
    nchunks = rows_per_worker // chunk

    def body(c, carry):
        base = wid * rows_per_worker + c * chunk
        pltpu.sync_copy(idx_hbm.at[pl.ds(base, chunk)], idx_v)
        pltpu.async_copy(kv_hbm.at[idx_v], rows_v, sem).wait()
        pltpu.sync_copy(rows_v, out_hbm.at[pl.ds(base, chunk)])
        return carry

    lax.fori_loop(0, nchunks, body, 0)


def _attn_body(q_ref, g_ref, w1_ref, b1_ref, w2_ref, b2_ref, out_ref, *, d):
    q = q_ref[0]
    g = g_ref[0]
    kfeat = g[:, :, :d]
    vfeat = g[:, :, d:]
    scores = jnp.sum(q[:, None, :] * kfeat, axis=-1) / math.sqrt(d)
    m = jnp.max(scores, axis=-1, keepdims=True)
    e = jnp.exp(scores - m)
    w = e / jnp.sum(e, axis=-1, keepdims=True)
    ctx = jnp.sum(w[:, :, None] * vfeat, axis=1)
    hid = jnp.dot(ctx, w1_ref[...],
                  preferred_element_type=jnp.float32) + b1_ref[...]
    hid = 0.5 * hid * (1.0 + lax.erf(hid / math.sqrt(2.0)))
    out_ref[0] = jnp.dot(hid, w2_ref[...],
                         preferred_element_type=jnp.float32) + b2_ref[...]


def kernel(h_e, x_orig, Wq, bq, Wk, bk, Wv, bv, W1, b1, W2, b2):
    b, T, n, d = h_e.shape
    t0 = T - 1
    t_start = max(0, t0 - _TAU + 1)
    tau = t0 - t_start + 1
    k = min(_K, n)
    rows = _ROWS
    nb = n // rows
    kt = k * tau

    x_last = x_orig[t0]
    wind = x_last[:, :, 4:6]
    windc = wind
    windr = jnp.transpose(wind, (0, 2, 1))
    h_last = h_e[:, t0]
    hist = h_e[:, t_start:t0 + 1].reshape(b, tau * n, d)

    full = lambda shp: pl.BlockSpec(shp, lambda *_: (0,) * len(shp))

    q, kv = pl.pallas_call(
        _qkv_body,
        grid=(b,),
        in_specs=[
            pl.BlockSpec((1, n, d), lambda i: (i, 0, 0)),
            pl.BlockSpec((1, tau * n, d), lambda i: (i, 0, 0)),
            full((d, d)), full((d,)), full((d, d)), full((d,)),
            full((d, d)), full((d,)),
        ],
        out_specs=[
            pl.BlockSpec((1, n, d), lambda i: (i, 0, 0)),
            pl.BlockSpec((1, tau * n, 2 * d), lambda i: (i, 0, 0)),
        ],
        out_shape=[
            jax.ShapeDtypeStruct((b, n, d), jnp.float32),
            jax.ShapeDtypeStruct((b, tau * n, 2 * d), jnp.float32),
        ],
    )(h_last, hist, Wq, bq, Wk, bk, Wv, bv)

    fidx = pl.pallas_call(
        functools.partial(_topk_body, n=n, tau=tau, k=k),
        grid=(b, nb),
        in_specs=[
            pl.BlockSpec((1, rows, 2), lambda i, r: (i, r, 0)),
            pl.BlockSpec((1, 2, n), lambda i, r: (i, 0, 0)),
        ],
        out_specs=pl.BlockSpec((1, rows, kt), lambda i, r: (i, r, 0)),
        out_shape=jax.ShapeDtypeStruct((b, n, kt), jnp.int32),
    )(windc, windr)

    total = b * n * kt
    try:
        info = plsc.get_sparse_core_info()
        num_cores, num_subcores = info.num_cores, info.num_subcores
    except ValueError:
        num_cores, num_subcores = 2, 16
    nw = num_cores * num_subcores
    rpw = total // nw
    chunk = 128
    mesh = plsc.VectorSubcoreMesh(core_axis_name="c", subcore_axis_name="s",
                                  num_cores=num_cores,
                                  num_subcores=num_subcores)
    g = pl.kernel(
        functools.partial(_sc_gather_body, rows_per_worker=rpw, chunk=chunk,
                          num_cores=num_cores),
        out_type=jax.ShapeDtypeStruct((total, 2 * d), jnp.float32),
        mesh=mesh,
        scratch_types=[
            pltpu.VMEM((chunk,), jnp.int32),
            pltpu.VMEM((chunk, 2 * d), jnp.float32),
            pltpu.SemaphoreType.DMA,
        ],
    )(kv.reshape(b * tau * n, 2 * d), fidx.reshape(total))

    out = pl.pallas_call(
        functools.partial(_attn_body, d=d),
        grid=(b, nb),
        in_specs=[
            pl.BlockSpec((1, rows, d), lambda i, r: (i, r, 0)),
            pl.BlockSpec((1, rows, kt, 2 * d), lambda i, r: (i, r, 0, 0)),
            full((d, d)), full((d,)), full((d, d)), full((d,)),
        ],
        out_specs=pl.BlockSpec((1, rows, d), lambda i, r: (i, r, 0)),
        out_shape=jax.ShapeDtypeStruct((b, n, d), jnp.float32),
    )(q, g.reshape(b, n, kt, 2 * d), W1, b1, W2, b2)
    return out
